# baseline (speedup 1.0000x reference)
import functools

import jax
import jax.numpy as jnp
from jax import lax
from jax.experimental import pallas as pl
from jax.experimental.pallas import tpu as pltpu
from jax.experimental.pallas import tpu_sc as plsc

B = 16384
D = 16
H = 16
NC, NS = 2, 16
NW = NC * NS
BPW = B // NW

_sc_mesh = plsc.VectorSubcoreMesh(
    core_axis_name="c", subcore_axis_name="s", num_cores=NC, num_subcores=NS)


@functools.partial(
    pl.kernel,
    out_type=[jax.ShapeDtypeStruct((B, D), jnp.float32)] * 4,
    mesh=_sc_mesh,
    scratch_types=[
        pltpu.VMEM((BPW,), jnp.int32),
        pltpu.VMEM((BPW,), jnp.int32),
        pltpu.SemaphoreType.DMA,
        pltpu.SemaphoreType.DMA,
        pltpu.SemaphoreType.DMA,
        pltpu.SemaphoreType.DMA,
    ],
    compiler_params=pltpu.CompilerParams(use_tc_tiling_on_sc=True),
)
def _sc_gather(t0, t1, t2, t3, u_h, i_h, o0, o1, o2, o3,
               uvm, ivm, s0, s1, s2, s3):
    wid = lax.axis_index("s") * NC + lax.axis_index("c")
    base = wid * BPW
    pltpu.sync_copy(u_h.at[pl.ds(base, BPW)], uvm)
    pltpu.sync_copy(i_h.at[pl.ds(base, BPW)], ivm)

    def issue(g, _):
        u16 = uvm[pl.ds(g * 16, 16)]
        i16 = ivm[pl.ds(g * 16, 16)]
        for k in range(16):
            u = u16[k]
            iv = i16[k]
            j = base + g * 16 + k
            pltpu.async_copy(t0.at[pl.ds(u, 1)], o0.at[pl.ds(j, 1)], s0)
            pltpu.async_copy(t1.at[pl.ds(iv, 1)], o1.at[pl.ds(j, 1)], s1)
            pltpu.async_copy(t2.at[pl.ds(u, 1)], o2.at[pl.ds(j, 1)], s2)
            pltpu.async_copy(t3.at[pl.ds(iv, 1)], o3.at[pl.ds(j, 1)], s3)
        return 0

    lax.fori_loop(0, BPW // 16, issue, 0)

    def drain(j, _):
        pltpu.make_async_copy(t0.at[pl.ds(0, 1)], o0.at[pl.ds(base, 1)], s0).wait()
        pltpu.make_async_copy(t1.at[pl.ds(0, 1)], o1.at[pl.ds(base, 1)], s1).wait()
        pltpu.make_async_copy(t2.at[pl.ds(0, 1)], o2.at[pl.ds(base, 1)], s2).wait()
        pltpu.make_async_copy(t3.at[pl.ds(0, 1)], o3.at[pl.ds(base, 1)], s3).wait()
        return 0

    lax.fori_loop(0, BPW, drain, 0)


BLK = 2048


def _tc_body(ug, ig, um, im, w1t, b1, wot, bo, out):
    gmf = ug[...] * ig[...]
    mlp_in = jnp.concatenate([um[...], im[...]], axis=1)
    h = jnp.maximum(
        jnp.dot(mlp_in, w1t[...], preferred_element_type=jnp.float32) + b1[...],
        0.0)
    x = jnp.concatenate([gmf, h], axis=1)
    logit = jnp.dot(x, wot[...], preferred_element_type=jnp.float32) + bo[...]
    out[...] = 1.0 / (1.0 + jnp.exp(-logit))


_tc_mlp = pl.pallas_call(
    _tc_body,
    grid=(B // BLK,),
    in_specs=[
        pl.BlockSpec((BLK, D), lambda b: (b, 0)),
        pl.BlockSpec((BLK, D), lambda b: (b, 0)),
        pl.BlockSpec((BLK, D), lambda b: (b, 0)),
        pl.BlockSpec((BLK, D), lambda b: (b, 0)),
        pl.BlockSpec((2 * D, H), lambda b: (0, 0)),
        pl.BlockSpec((1, H), lambda b: (0, 0)),
        pl.BlockSpec((D + H, 1), lambda b: (0, 0)),
        pl.BlockSpec((1, 1), lambda b: (0, 0)),
    ],
    out_specs=pl.BlockSpec((BLK, 1), lambda b: (b, 0)),
    out_shape=jax.ShapeDtypeStruct((B, 1), jnp.float32),
)


def kernel(user_gmf, item_gmf, user_mlp, item_mlp, W1, b1, Wo, bo, u, i):
    ug, ig, um, im = _sc_gather(user_gmf, item_gmf, user_mlp, item_mlp, u, i)
    out = _tc_mlp(ug, ig, um, im, W1.T, b1.reshape(1, H), Wo.T,
                  bo.reshape(1, 1))
    return out.reshape(B)


# speedup vs baseline: 1.3493x; 1.3493x over previous
import functools

import jax
import jax.numpy as jnp
from jax import lax
from jax.experimental import pallas as pl
from jax.experimental.pallas import tpu as pltpu
from jax.experimental.pallas import tpu_sc as plsc

B = 16384
D = 16
H = 16
NROW = 1000000
NL = NROW // 8
NC, NS = 2, 16
NW = NC * NS
BPW = B // NW
G = BPW // 16

_sc_mesh = plsc.VectorSubcoreMesh(
    core_axis_name="c", subcore_axis_name="s", num_cores=NC, num_subcores=NS)


@functools.partial(
    pl.kernel,
    out_type=[jax.ShapeDtypeStruct((D, B), jnp.float32)] * 4,
    mesh=_sc_mesh,
    scratch_types=[
        pltpu.VMEM((BPW,), jnp.int32),
        pltpu.VMEM((BPW,), jnp.int32),
        pltpu.VMEM((BPW,), jnp.int32),
        pltpu.VMEM((BPW,), jnp.int32),
        pltpu.VMEM((BPW, 128), jnp.float32),
        pltpu.VMEM((4, D, BPW), jnp.float32),
        pltpu.SemaphoreType.DMA,
    ],
    compiler_params=pltpu.CompilerParams(
        use_tc_tiling_on_sc=False, needs_layout_passes=False),
)
def _sc_gather(c0, c1, c2, c3, qu_h, qi_h, lu_h, li_h,
               o0, o1, o2, o3, qu, qi, lu, li, rows, sel, sem):
    wid = lax.axis_index("s") * NC + lax.axis_index("c")
    base = wid * BPW
    pltpu.sync_copy(qu_h.at[pl.ds(base, BPW)], qu)
    pltpu.sync_copy(qi_h.at[pl.ds(base, BPW)], qi)
    pltpu.sync_copy(lu_h.at[pl.ds(base, BPW)], lu)
    pltpu.sync_copy(li_h.at[pl.ds(base, BPW)], li)
    lane = lax.iota(jnp.int32, 16)

    for t, (tbl, q, l) in enumerate((
            (c0, qu, lu), (c1, qi, li), (c2, qu, lu), (c3, qi, li))):
        pltpu.async_copy(tbl.at[q], rows, sem).wait()

        def body(g, _, l=l, t=t):
            j16 = g * 16 + lane
            l16 = l[pl.ds(g * 16, 16)]
            for d in range(D):
                v = plsc.load_gather(rows, [j16, l16 + d])
                sel[t, d, pl.ds(g * 16, 16)] = v
            return 0

        lax.fori_loop(0, G, body, 0)

    for t, o in enumerate((o0, o1, o2, o3)):
        pltpu.sync_copy(sel.at[t], o.at[:, pl.ds(base, BPW)])


BLK = 4096


def _tc_body(ug, ig, um, im, w1, b1, wo, bo, out):
    gmf = ug[...] * ig[...]
    x = jnp.concatenate([um[...], im[...]], axis=0)
    h = jnp.maximum(
        jnp.dot(w1[...], x, preferred_element_type=jnp.float32) + b1[...],
        0.0)
    xc = jnp.concatenate([gmf, h], axis=0)
    logit = jnp.dot(wo[...], xc, preferred_element_type=jnp.float32) + bo[...]
    out[...] = 1.0 / (1.0 + jnp.exp(-logit))


_tc_mlp = pl.pallas_call(
    _tc_body,
    grid=(B // BLK,),
    in_specs=[
        pl.BlockSpec((D, BLK), lambda b: (0, b)),
        pl.BlockSpec((D, BLK), lambda b: (0, b)),
        pl.BlockSpec((D, BLK), lambda b: (0, b)),
        pl.BlockSpec((D, BLK), lambda b: (0, b)),
        pl.BlockSpec((H, 2 * D), lambda b: (0, 0)),
        pl.BlockSpec((H, 1), lambda b: (0, 0)),
        pl.BlockSpec((1, D + H), lambda b: (0, 0)),
        pl.BlockSpec((1, 1), lambda b: (0, 0)),
    ],
    out_specs=pl.BlockSpec((1, BLK), lambda b: (0, b)),
    out_shape=jax.ShapeDtypeStruct((1, B), jnp.float32),
)


def kernel(user_gmf, item_gmf, user_mlp, item_mlp, W1, b1, Wo, bo, u, i):
    cug = user_gmf.reshape(NL, 128)
    cig = item_gmf.reshape(NL, 128)
    cum = user_mlp.reshape(NL, 128)
    cim = item_mlp.reshape(NL, 128)
    qu = jnp.right_shift(u, 3)
    qi = jnp.right_shift(i, 3)
    lu = jnp.left_shift(jnp.bitwise_and(u, 7), 4)
    li = jnp.left_shift(jnp.bitwise_and(i, 7), 4)
    ug, ig, um, im = _sc_gather(cug, cig, cum, cim, qu, qi, lu, li)
    out = _tc_mlp(ug, ig, um, im, W1, b1.reshape(H, 1), Wo,
                  bo.reshape(1, 1))
    return out.reshape(B)


# speedup vs baseline: 1.3513x; 1.0015x over previous
import functools

import jax
import jax.numpy as jnp
from jax import lax
from jax.experimental import pallas as pl
from jax.experimental.pallas import tpu as pltpu
from jax.experimental.pallas import tpu_sc as plsc

B = 16384
D = 16
H = 16
NROW = 1000000
NL = NROW // 8
NC, NS = 2, 16
NW = NC * NS
BPW = B // NW
G = BPW // 16

_sc_mesh = plsc.VectorSubcoreMesh(
    core_axis_name="c", subcore_axis_name="s", num_cores=NC, num_subcores=NS)

CPW = (NL // NW) // 8 * 8
REM = NL - CPW * NW


@functools.partial(
    pl.kernel,
    out_type=[jax.ShapeDtypeStruct((NL, 128), jnp.float32)] * 4,
    mesh=_sc_mesh,
    scratch_types=[pltpu.SemaphoreType.DMA],
    compiler_params=pltpu.CompilerParams(use_tc_tiling_on_sc=True),
)
def _sc_compact(t0, t1, t2, t3, c0, c1, c2, c3, sem):
    wid = lax.axis_index("s") * NC + lax.axis_index("c")
    s = pl.multiple_of(wid * CPW, 8)
    cps = []
    for tb, cm in ((t0, c0), (t1, c1), (t2, c2), (t3, c3)):
        dst = cm.at[pl.ds(s, CPW)].reshape(CPW, 8, D)
        cps.append(pltpu.async_copy(tb.at[pl.ds(s, CPW)], dst, sem))
    for cp in cps:
        cp.wait()

    @pl.when(wid == 0)
    def _():
        tail = []
        for tb, cm in ((t0, c0), (t1, c1), (t2, c2), (t3, c3)):
            dst = cm.at[pl.ds(NL - REM, REM)].reshape(REM, 8, D)
            tail.append(pltpu.async_copy(tb.at[pl.ds(NL - REM, REM)], dst, sem))
        for cp in tail:
            cp.wait()


@functools.partial(
    pl.kernel,
    out_type=[jax.ShapeDtypeStruct((D, B), jnp.float32)] * 4,
    mesh=_sc_mesh,
    scratch_types=[
        pltpu.VMEM((BPW,), jnp.int32),
        pltpu.VMEM((BPW,), jnp.int32),
        pltpu.VMEM((BPW,), jnp.int32),
        pltpu.VMEM((BPW,), jnp.int32),
        pltpu.VMEM((BPW, 128), jnp.float32),
        pltpu.VMEM((4, D, BPW), jnp.float32),
        pltpu.SemaphoreType.DMA,
    ],
    compiler_params=pltpu.CompilerParams(
        use_tc_tiling_on_sc=False, needs_layout_passes=False,
        disable_bounds_checks=True, disable_semaphore_checks=True,
        skip_device_barrier=True),
)
def _sc_gather(c0, c1, c2, c3, qu_h, qi_h, lu_h, li_h,
               o0, o1, o2, o3, qu, qi, lu, li, rows, sel, sem):
    wid = lax.axis_index("s") * NC + lax.axis_index("c")
    base = wid * BPW
    pltpu.sync_copy(qu_h.at[pl.ds(base, BPW)], qu)
    pltpu.sync_copy(qi_h.at[pl.ds(base, BPW)], qi)
    pltpu.sync_copy(lu_h.at[pl.ds(base, BPW)], lu)
    pltpu.sync_copy(li_h.at[pl.ds(base, BPW)], li)
    lane = lax.iota(jnp.int32, 16)

    for t, (tbl, q, l) in enumerate((
            (c0, qu, lu), (c1, qi, li), (c2, qu, lu), (c3, qi, li))):
        pltpu.async_copy(tbl.at[q], rows, sem).wait()

        def body(g, _, l=l, t=t):
            j16 = g * 16 + lane
            l16 = l[pl.ds(g * 16, 16)]
            for d in range(D):
                v = plsc.load_gather(rows, [j16, l16 + d])
                sel[t, d, pl.ds(g * 16, 16)] = v
            return 0

        lax.fori_loop(0, G, body, 0)

    for t, o in enumerate((o0, o1, o2, o3)):
        pltpu.sync_copy(sel.at[t], o.at[:, pl.ds(base, BPW)])


BLK = 4096


def _tc_body(ug, ig, um, im, w1, b1, wo, bo, out):
    gmf = ug[...] * ig[...]
    x = jnp.concatenate([um[...], im[...]], axis=0)
    h = jnp.maximum(
        jnp.dot(w1[...], x, preferred_element_type=jnp.float32) + b1[...],
        0.0)
    xc = jnp.concatenate([gmf, h], axis=0)
    logit = jnp.dot(wo[...], xc, preferred_element_type=jnp.float32) + bo[...]
    out[...] = 1.0 / (1.0 + jnp.exp(-logit))


_tc_mlp = pl.pallas_call(
    _tc_body,
    grid=(B // BLK,),
    in_specs=[
        pl.BlockSpec((D, BLK), lambda b: (0, b)),
        pl.BlockSpec((D, BLK), lambda b: (0, b)),
        pl.BlockSpec((D, BLK), lambda b: (0, b)),
        pl.BlockSpec((D, BLK), lambda b: (0, b)),
        pl.BlockSpec((H, 2 * D), lambda b: (0, 0)),
        pl.BlockSpec((H, 1), lambda b: (0, 0)),
        pl.BlockSpec((1, D + H), lambda b: (0, 0)),
        pl.BlockSpec((1, 1), lambda b: (0, 0)),
    ],
    out_specs=pl.BlockSpec((1, BLK), lambda b: (0, b)),
    out_shape=jax.ShapeDtypeStruct((1, B), jnp.float32),
)


def kernel(user_gmf, item_gmf, user_mlp, item_mlp, W1, b1, Wo, bo, u, i):
    cug = user_gmf.reshape(NL, 128)
    cig = item_gmf.reshape(NL, 128)
    cum = user_mlp.reshape(NL, 128)
    cim = item_mlp.reshape(NL, 128)
    qu = jnp.right_shift(u, 3)
    qi = jnp.right_shift(i, 3)
    lu = jnp.left_shift(jnp.bitwise_and(u, 7), 4)
    li = jnp.left_shift(jnp.bitwise_and(i, 7), 4)
    ug, ig, um, im = _sc_gather(cug, cig, cum, cim, qu, qi, lu, li)
    out = _tc_mlp(ug, ig, um, im, W1, b1.reshape(H, 1), Wo,
                  bo.reshape(1, 1))
    return out.reshape(B)


# speedup vs baseline: 1.3598x; 1.0062x over previous
import functools

import jax
import jax.numpy as jnp
from jax import lax
from jax.experimental import pallas as pl
from jax.experimental.pallas import tpu as pltpu
from jax.experimental.pallas import tpu_sc as plsc

B = 16384
D = 16
H = 16
NROW = 1000000
NL = NROW // 8
NC, NS = 2, 16
NW = NC * NS
BPW = B // NW
G = BPW // 16

_sc_mesh = plsc.VectorSubcoreMesh(
    core_axis_name="c", subcore_axis_name="s", num_cores=NC, num_subcores=NS)

CPW = (NL // NW) // 8 * 8
REM = NL - CPW * NW


@functools.partial(
    pl.kernel,
    out_type=[jax.ShapeDtypeStruct((NL, 128), jnp.float32)] * 4,
    mesh=_sc_mesh,
    scratch_types=[pltpu.SemaphoreType.DMA],
    compiler_params=pltpu.CompilerParams(use_tc_tiling_on_sc=True),
)
def _sc_compact(t0, t1, t2, t3, c0, c1, c2, c3, sem):
    wid = lax.axis_index("s") * NC + lax.axis_index("c")
    s = pl.multiple_of(wid * CPW, 8)
    cps = []
    for tb, cm in ((t0, c0), (t1, c1), (t2, c2), (t3, c3)):
        dst = cm.at[pl.ds(s, CPW)].reshape(CPW, 8, D)
        cps.append(pltpu.async_copy(tb.at[pl.ds(s, CPW)], dst, sem))
    for cp in cps:
        cp.wait()

    @pl.when(wid == 0)
    def _():
        tail = []
        for tb, cm in ((t0, c0), (t1, c1), (t2, c2), (t3, c3)):
            dst = cm.at[pl.ds(NL - REM, REM)].reshape(REM, 8, D)
            tail.append(pltpu.async_copy(tb.at[pl.ds(NL - REM, REM)], dst, sem))
        for cp in tail:
            cp.wait()


@functools.partial(
    pl.kernel,
    out_type=[jax.ShapeDtypeStruct((D, B), jnp.float32)] * 4,
    mesh=_sc_mesh,
    scratch_types=[
        pltpu.VMEM((BPW,), jnp.int32),
        pltpu.VMEM((BPW,), jnp.int32),
        pltpu.VMEM((BPW,), jnp.int32),
        pltpu.VMEM((BPW,), jnp.int32),
        pltpu.VMEM((BPW, 128), jnp.float32),
        pltpu.VMEM((4, D, BPW), jnp.float32),
        pltpu.SemaphoreType.DMA,
    ],
    compiler_params=pltpu.CompilerParams(
        use_tc_tiling_on_sc=True, needs_layout_passes=False),
)
def _sc_gather(c0, c1, c2, c3, qu_h, qi_h, lu_h, li_h,
               o0, o1, o2, o3, qu, qi, lu, li, rows, sel, sem):
    wid = lax.axis_index("s") * NC + lax.axis_index("c")
    base = wid * BPW
    pltpu.sync_copy(qu_h.at[pl.ds(base, BPW)], qu)
    pltpu.sync_copy(qi_h.at[pl.ds(base, BPW)], qi)
    pltpu.sync_copy(lu_h.at[pl.ds(base, BPW)], lu)
    pltpu.sync_copy(li_h.at[pl.ds(base, BPW)], li)
    lane = lax.iota(jnp.int32, 16)

    for t, (tbl, q, l) in enumerate((
            (c0, qu, lu), (c1, qi, li), (c2, qu, lu), (c3, qi, li))):
        pltpu.async_copy(tbl.at[q], rows, sem).wait()

        def body(g, _, l=l, t=t):
            j16 = g * 16 + lane
            l16 = l[pl.ds(g * 16, 16)]
            for d in range(D):
                v = plsc.load_gather(rows, [j16, l16 + d])
                sel[t, d, pl.ds(g * 16, 16)] = v
            return 0

        lax.fori_loop(0, G, body, 0)

    for t, o in enumerate((o0, o1, o2, o3)):
        pltpu.sync_copy(sel.at[t], o.at[:, pl.ds(base, BPW)])


BLK = 4096


def _tc_body(ug, ig, um, im, w1, b1, wo, bo, out):
    gmf = ug[...] * ig[...]
    x = jnp.concatenate([um[...], im[...]], axis=0)
    h = jnp.maximum(
        jnp.dot(w1[...], x, preferred_element_type=jnp.float32) + b1[...],
        0.0)
    xc = jnp.concatenate([gmf, h], axis=0)
    logit = jnp.dot(wo[...], xc, preferred_element_type=jnp.float32) + bo[...]
    out[...] = 1.0 / (1.0 + jnp.exp(-logit))


_tc_mlp = pl.pallas_call(
    _tc_body,
    grid=(B // BLK,),
    in_specs=[
        pl.BlockSpec((D, BLK), lambda b: (0, b)),
        pl.BlockSpec((D, BLK), lambda b: (0, b)),
        pl.BlockSpec((D, BLK), lambda b: (0, b)),
        pl.BlockSpec((D, BLK), lambda b: (0, b)),
        pl.BlockSpec((H, 2 * D), lambda b: (0, 0)),
        pl.BlockSpec((H, 1), lambda b: (0, 0)),
        pl.BlockSpec((1, D + H), lambda b: (0, 0)),
        pl.BlockSpec((1, 1), lambda b: (0, 0)),
    ],
    out_specs=pl.BlockSpec((1, BLK), lambda b: (0, b)),
    out_shape=jax.ShapeDtypeStruct((1, B), jnp.float32),
)


def kernel(user_gmf, item_gmf, user_mlp, item_mlp, W1, b1, Wo, bo, u, i):
    cug = user_gmf.reshape(NL, 128)
    cig = item_gmf.reshape(NL, 128)
    cum = user_mlp.reshape(NL, 128)
    cim = item_mlp.reshape(NL, 128)
    qu = jnp.right_shift(u, 3)
    qi = jnp.right_shift(i, 3)
    lu = jnp.left_shift(jnp.bitwise_and(u, 7), 4)
    li = jnp.left_shift(jnp.bitwise_and(i, 7), 4)
    ug, ig, um, im = _sc_gather(cug, cig, cum, cim, qu, qi, lu, li)
    out = _tc_mlp(ug, ig, um, im, W1, b1.reshape(H, 1), Wo,
                  bo.reshape(1, 1))
    return out.reshape(B)


# speedup vs baseline: 1.6529x; 1.2156x over previous
import functools

import jax
import jax.numpy as jnp
from jax import lax
from jax.experimental import pallas as pl
from jax.experimental.pallas import tpu as pltpu
from jax.experimental.pallas import tpu_sc as plsc

B = 16384
D = 16
H = 16
NROW = 1000000
NL = NROW // 8
NC, NS = 2, 16
NW = NC * NS
BPW = B // NW
G = BPW // 16

_sc_mesh = plsc.VectorSubcoreMesh(
    core_axis_name="c", subcore_axis_name="s", num_cores=NC, num_subcores=NS)

CPW = (NL // NW) // 8 * 8
REM = NL - CPW * NW


@functools.partial(
    pl.kernel,
    out_type=[jax.ShapeDtypeStruct((NL, 128), jnp.float32)] * 4,
    mesh=_sc_mesh,
    scratch_types=[pltpu.SemaphoreType.DMA],
    compiler_params=pltpu.CompilerParams(use_tc_tiling_on_sc=True),
)
def _sc_compact(t0, t1, t2, t3, c0, c1, c2, c3, sem):
    wid = lax.axis_index("s") * NC + lax.axis_index("c")
    s = pl.multiple_of(wid * CPW, 8)
    cps = []
    for tb, cm in ((t0, c0), (t1, c1), (t2, c2), (t3, c3)):
        dst = cm.at[pl.ds(s, CPW)].reshape(CPW, 8, D)
        cps.append(pltpu.async_copy(tb.at[pl.ds(s, CPW)], dst, sem))
    for cp in cps:
        cp.wait()

    @pl.when(wid == 0)
    def _():
        tail = []
        for tb, cm in ((t0, c0), (t1, c1), (t2, c2), (t3, c3)):
            dst = cm.at[pl.ds(NL - REM, REM)].reshape(REM, 8, D)
            tail.append(pltpu.async_copy(tb.at[pl.ds(NL - REM, REM)], dst, sem))
        for cp in tail:
            cp.wait()


@functools.partial(
    pl.kernel,
    out_type=[jax.ShapeDtypeStruct((D, B), jnp.float32)] * 4,
    mesh=_sc_mesh,
    scratch_types=[
        pltpu.VMEM((BPW,), jnp.int32),
        pltpu.VMEM((BPW,), jnp.int32),
        pltpu.VMEM((BPW,), jnp.int32),
        pltpu.VMEM((BPW,), jnp.int32),
        pltpu.VMEM((BPW, 128), jnp.float32),
        pltpu.VMEM((4, D, BPW), jnp.float32),
        pltpu.SemaphoreType.DMA,
    ],
    compiler_params=pltpu.CompilerParams(
        use_tc_tiling_on_sc=True, needs_layout_passes=False),
)
def _sc_gather(c0, c1, c2, c3, qu_h, qi_h, lu_h, li_h,
               o0, o1, o2, o3, qu, qi, lu, li, rows, sel, sem):
    wid = lax.axis_index("s") * NC + lax.axis_index("c")
    base = wid * BPW
    pltpu.sync_copy(qu_h.at[pl.ds(base, BPW)], qu)
    pltpu.sync_copy(qi_h.at[pl.ds(base, BPW)], qi)
    pltpu.sync_copy(lu_h.at[pl.ds(base, BPW)], lu)
    pltpu.sync_copy(li_h.at[pl.ds(base, BPW)], li)
    lane = lax.iota(jnp.int32, 16)

    for t, (tbl, q, l) in enumerate((
            (c0, qu, lu), (c1, qi, li), (c2, qu, lu), (c3, qi, li))):
        pltpu.async_copy(tbl.at[q], rows, sem).wait()

        def body(g, _, l=l, t=t):
            j16 = g * 16 + lane
            l16 = l[pl.ds(g * 16, 16)]
            for d in range(D):
                v = plsc.load_gather(rows, [j16, l16 + d])
                sel[t, d, pl.ds(g * 16, 16)] = v
            return 0

        lax.fori_loop(0, G, body, 0)

    for t, o in enumerate((o0, o1, o2, o3)):
        pltpu.sync_copy(sel.at[t], o.at[:, pl.ds(base, BPW)])


RB = 1000


def _repack_body(x, out):
    out[...] = x[...].reshape(RB, 128)


_tc_repack = pl.pallas_call(
    _repack_body,
    grid=(NL // RB,),
    in_specs=[pl.BlockSpec((RB, 8, D), lambda b: (b, 0, 0))],
    out_specs=pl.BlockSpec((RB, 128), lambda b: (b, 0)),
    out_shape=jax.ShapeDtypeStruct((NL, 128), jnp.float32),
)


BLK = 4096


def _tc_body(ug, ig, um, im, w1, b1, wo, bo, out):
    gmf = ug[...] * ig[...]
    x = jnp.concatenate([um[...], im[...]], axis=0)
    h = jnp.maximum(
        jnp.dot(w1[...], x, preferred_element_type=jnp.float32) + b1[...],
        0.0)
    xc = jnp.concatenate([gmf, h], axis=0)
    logit = jnp.dot(wo[...], xc, preferred_element_type=jnp.float32) + bo[...]
    out[...] = 1.0 / (1.0 + jnp.exp(-logit))


_tc_mlp = pl.pallas_call(
    _tc_body,
    grid=(B // BLK,),
    in_specs=[
        pl.BlockSpec((D, BLK), lambda b: (0, b)),
        pl.BlockSpec((D, BLK), lambda b: (0, b)),
        pl.BlockSpec((D, BLK), lambda b: (0, b)),
        pl.BlockSpec((D, BLK), lambda b: (0, b)),
        pl.BlockSpec((H, 2 * D), lambda b: (0, 0)),
        pl.BlockSpec((H, 1), lambda b: (0, 0)),
        pl.BlockSpec((1, D + H), lambda b: (0, 0)),
        pl.BlockSpec((1, 1), lambda b: (0, 0)),
    ],
    out_specs=pl.BlockSpec((1, BLK), lambda b: (0, b)),
    out_shape=jax.ShapeDtypeStruct((1, B), jnp.float32),
)


def kernel(user_gmf, item_gmf, user_mlp, item_mlp, W1, b1, Wo, bo, u, i):
    cug = _tc_repack(user_gmf.reshape(NL, 8, D))
    cig = _tc_repack(item_gmf.reshape(NL, 8, D))
    cum = _tc_repack(user_mlp.reshape(NL, 8, D))
    cim = _tc_repack(item_mlp.reshape(NL, 8, D))
    qu = jnp.right_shift(u, 3)
    qi = jnp.right_shift(i, 3)
    lu = jnp.left_shift(jnp.bitwise_and(u, 7), 4)
    li = jnp.left_shift(jnp.bitwise_and(i, 7), 4)
    ug, ig, um, im = _sc_gather(cug, cig, cum, cim, qu, qi, lu, li)
    out = _tc_mlp(ug, ig, um, im, W1, b1.reshape(H, 1), Wo,
                  bo.reshape(1, 1))
    return out.reshape(B)
